# baseline (speedup 1.0000x reference)
import functools

import jax
import jax.numpy as jnp
from jax import lax
from jax.experimental import pallas as pl
from jax.experimental.pallas import tpu as pltpu
from jax.experimental.pallas import tpu_sc as plsc

KNN = 27
KP = 28
DIM = 128
NPTS = 2048
TP = 256
TQ = 512


def _topk_body(xvt_ref, xvt_t_ref, ids_ref):
    b = pl.program_id(0)
    xvt = xvt_ref[0]
    xvt_t = xvt_t_ref[0]
    g = lax.dot_general(xvt, xvt_t, (((1,), (0,)), ((), ())),
                        preferred_element_type=jnp.float32)
    sqt = jnp.sum(xvt * xvt, axis=1)
    sqa = jnp.sum(xvt_t * xvt_t, axis=0)
    d = sqt[:, None] + sqa[None, :] - 2.0 * g
    qio = lax.broadcasted_iota(jnp.int32, (TP, NPTS), 1)
    kio = lax.broadcasted_iota(jnp.int32, (TP, KP), 1)
    rowbase = b * NPTS * KNN
    acc = jnp.zeros((TP, KP), jnp.int32)
    for k in range(KNN):
        m = jnp.min(d, axis=1)
        amin = jnp.min(jnp.where(d == m[:, None], qio, NPTS), axis=1)
        d = jnp.where(qio == amin[:, None], jnp.float32(jnp.inf), d)
        rowid = amin * KNN + (rowbase + k)
        acc = jnp.where(kio == k, rowid[:, None], acc)
    ids_ref[0] = acc


def _topk(xvp, xvt_t):
    nb = xvp.shape[0]
    return pl.pallas_call(
        _topk_body,
        grid=(nb, NPTS // TP),
        in_specs=[
            pl.BlockSpec((1, TP, 8), lambda b, t: (b, t, 0)),
            pl.BlockSpec((1, 8, NPTS), lambda b, t: (b, 0, 0)),
        ],
        out_specs=pl.BlockSpec((1, TP, KP), lambda b, t: (b, t, 0)),
        out_shape=jax.ShapeDtypeStruct((nb, NPTS, KP), jnp.int32),
    )(xvp, xvt_t)


def _zmat_body(x_ref, w2_ref, z_ref):
    z_ref[0] = jnp.dot(x_ref[0], w2_ref[...],
                       preferred_element_type=jnp.float32)


def _zmat(x, w2):
    nb = x.shape[0]
    return pl.pallas_call(
        _zmat_body,
        grid=(nb, NPTS // TQ),
        in_specs=[
            pl.BlockSpec((1, TQ, DIM), lambda b, t: (b, t, 0)),
            pl.BlockSpec((DIM, KNN * DIM), lambda b, t: (0, 0)),
        ],
        out_specs=pl.BlockSpec((1, TQ, KNN * DIM), lambda b, t: (b, t, 0)),
        out_shape=jax.ShapeDtypeStruct((nb, NPTS, KNN * DIM), jnp.float32),
    )(x, w2)


def _gather_sum(z3, idsf, bias, n_points):
    CP = 4
    CR = CP * KP
    NW = 32
    ppw = n_points // NW
    nchunks = ppw // CP
    NB = 4
    ngroups = nchunks // NB
    mesh = plsc.VectorSubcoreMesh(core_axis_name="c", subcore_axis_name="s")

    @functools.partial(
        pl.kernel,
        mesh=mesh,
        out_type=jax.ShapeDtypeStruct((n_points, DIM), jnp.float32),
        scratch_types=[
            pltpu.VMEM((ppw * KP,), jnp.int32),
            pltpu.VMEM((NB, CR, DIM), jnp.float32),
            pltpu.VMEM((NB, CP, DIM), jnp.float32),
            pltpu.VMEM((DIM,), jnp.float32),
        ] + [pltpu.SemaphoreType.DMA] * (2 * NB),
    )
    def body(z3_hbm, ids_hbm, b_hbm, out_hbm, idx_v, buf_v, ob_v, bias_v,
             *sems):
        sg = sems[:NB]
        so = sems[NB:]
        ncores = 2
        wid = lax.axis_index("s") * ncores + lax.axis_index("c")
        pstart = wid * ppw
        pltpu.sync_copy(b_hbm, bias_v)
        pltpu.sync_copy(ids_hbm.at[pl.ds(pstart * KP, ppw * KP)], idx_v)

        def g_issue(slot, c):
            pltpu.async_copy(z3_hbm.at[idx_v.at[pl.ds(c * CR, CR)]],
                             buf_v.at[slot], sg[slot])

        def g_wait(slot, c):
            pltpu.make_async_copy(z3_hbm.at[idx_v.at[pl.ds(c * CR, CR)]],
                                  buf_v.at[slot], sg[slot]).wait()

        def process(slot, c):
            p0 = pstart + c * CP
            @pl.when(c >= NB)
            def _():
                pltpu.make_async_copy(
                    ob_v.at[slot], out_hbm.at[pl.ds(p0 - NB * CP, CP)],
                    so[slot]).wait()
            for pt in range(CP):
                for cc in range(DIM // 16):
                    vals = [buf_v[slot, pt * KP + j, pl.ds(cc * 16, 16)]
                            for j in range(KNN)]
                    vals.append(bias_v[pl.ds(cc * 16, 16)])
                    while len(vals) > 1:
                        nxt = [vals[i] + vals[i + 1]
                               for i in range(0, len(vals) - 1, 2)]
                        if len(vals) % 2:
                            nxt.append(vals[-1])
                        vals = nxt
                    ob_v[slot, pt, pl.ds(cc * 16, 16)] = vals[0]
            pltpu.async_copy(ob_v.at[slot], out_hbm.at[pl.ds(p0, CP)],
                             so[slot])

        for s in range(NB - 1):
            g_issue(s, s)

        def group(g, carry):
            for s in range(NB):
                c = NB * g + s

                @pl.when(c + NB - 1 < nchunks)
                def _():
                    g_issue((s + NB - 1) % NB, c + NB - 1)
                g_wait(s, c)
                process(s, c)
            return carry

        lax.fori_loop(0, ngroups, group, 0)
        for s in range(NB):
            c = nchunks - NB + s
            pltpu.make_async_copy(
                ob_v.at[s], out_hbm.at[pl.ds(pstart + c * CP, CP)],
                so[s]).wait()

    return body(z3, idsf, bias)


def kernel(x, x_v, W, b):
    nb, npts, dim = x.shape
    xvp = jnp.pad(x_v, ((0, 0), (0, 0), (0, 5)))
    xvt_t = jnp.swapaxes(xvp, 1, 2)
    ids = _topk(xvp, xvt_t)
    w2 = W.reshape(DIM, KNN, DIM).transpose(2, 1, 0).reshape(DIM, KNN * DIM)
    z = _zmat(x, w2)
    z3 = z.reshape(nb * npts * KNN, dim)
    idsf = ids.reshape(nb * npts * KP)
    out = _gather_sum(z3, idsf, b, nb * npts)
    return out.reshape(nb, npts, dim)


# speedup vs baseline: 1.2362x; 1.2362x over previous
import functools

import jax
import jax.numpy as jnp
from jax import lax
from jax.experimental import pallas as pl
from jax.experimental.pallas import tpu as pltpu
from jax.experimental.pallas import tpu_sc as plsc

KNN = 27
KP = 28
DIM = 128
NPTS = 2048
TP = 256
TQ = 512


def _topk_body(xvt_ref, xvt_t_ref, ids_ref):
    b = pl.program_id(0)
    xvt = xvt_ref[0]
    xvt_t = xvt_t_ref[0]
    g = lax.dot_general(xvt, xvt_t, (((1,), (0,)), ((), ())),
                        preferred_element_type=jnp.float32)
    sqt = jnp.sum(xvt * xvt, axis=1)
    sqa = jnp.sum(xvt_t * xvt_t, axis=0)
    d = sqt[:, None] + sqa[None, :] - 2.0 * g
    qio = lax.broadcasted_iota(jnp.int32, (TP, NPTS), 1)
    kio = lax.broadcasted_iota(jnp.int32, (TP, KP), 1)
    rowbase = b * NPTS * KNN
    t = pl.program_id(1)
    acc = (rowbase + (t * TP) * KNN
           + lax.broadcasted_iota(jnp.int32, (TP, KP), 0) * KNN)
    for k in range(KNN):
        m = jnp.min(d, axis=1)
        amin = jnp.min(jnp.where(d == m[:, None], qio, NPTS), axis=1)
        d = jnp.where(qio == amin[:, None], jnp.float32(jnp.inf), d)
        rowid = amin * KNN + (rowbase + k)
        acc = jnp.where(kio == k, rowid[:, None], acc)
    ids_ref[0] = acc


def _topk(xvp, xvt_t):
    nb = xvp.shape[0]
    return pl.pallas_call(
        _topk_body,
        grid=(nb, NPTS // TP),
        in_specs=[
            pl.BlockSpec((1, TP, 8), lambda b, t: (b, t, 0)),
            pl.BlockSpec((1, 8, NPTS), lambda b, t: (b, 0, 0)),
        ],
        out_specs=pl.BlockSpec((1, TP, KP), lambda b, t: (b, t, 0)),
        out_shape=jax.ShapeDtypeStruct((nb, NPTS, KP), jnp.int32),
    )(xvp, xvt_t)


def _zmat_body(x_ref, w2_ref, z_ref):
    z_ref[0] = jnp.dot(x_ref[0], w2_ref[...],
                       preferred_element_type=jnp.float32)


def _zmat(x, w2):
    nb = x.shape[0]
    return pl.pallas_call(
        _zmat_body,
        grid=(nb, NPTS // TQ),
        in_specs=[
            pl.BlockSpec((1, TQ, DIM), lambda b, t: (b, t, 0)),
            pl.BlockSpec((DIM, KNN * DIM), lambda b, t: (0, 0)),
        ],
        out_specs=pl.BlockSpec((1, TQ, KNN * DIM), lambda b, t: (b, t, 0)),
        out_shape=jax.ShapeDtypeStruct((nb, NPTS, KNN * DIM), jnp.float32),
    )(x, w2)


def _gather_sum(z3, idsf, bias, n_points):
    CP = 4
    CR = CP * KP
    NW = 32
    ppw = n_points // NW
    nchunks = ppw // CP
    NB = 4
    ngroups = nchunks // NB
    mesh = plsc.VectorSubcoreMesh(core_axis_name="c", subcore_axis_name="s")

    @functools.partial(
        pl.kernel,
        mesh=mesh,
        out_type=jax.ShapeDtypeStruct((n_points, DIM), jnp.float32),
        scratch_types=[
            pltpu.VMEM((ppw * KP,), jnp.int32),
            pltpu.VMEM((NB, CR, DIM), jnp.float32),
            pltpu.VMEM((NB, CP, DIM), jnp.float32),
            pltpu.VMEM((DIM,), jnp.float32),
        ] + [pltpu.SemaphoreType.DMA] * (2 * NB),
    )
    def body(z3_hbm, ids_hbm, b_hbm, out_hbm, idx_v, buf_v, ob_v, bias_v,
             *sems):
        sg = sems[:NB]
        so = sems[NB:]
        ncores = 2
        wid = lax.axis_index("s") * ncores + lax.axis_index("c")
        pstart = wid * ppw
        pltpu.sync_copy(b_hbm, bias_v)
        pltpu.sync_copy(ids_hbm.at[pl.ds(pstart * KP, ppw * KP)], idx_v)

        def g_issue(slot, c):
            pltpu.async_copy(z3_hbm.at[idx_v.at[pl.ds(c * CR, CR)]],
                             buf_v.at[slot], sg[slot])

        def g_wait(slot, c):
            pltpu.make_async_copy(z3_hbm.at[idx_v.at[pl.ds(c * CR, CR)]],
                                  buf_v.at[slot], sg[slot]).wait()

        def process(slot, c):
            p0 = pstart + c * CP
            @pl.when(c >= NB)
            def _():
                pltpu.make_async_copy(
                    ob_v.at[slot], out_hbm.at[pl.ds(p0 - NB * CP, CP)],
                    so[slot]).wait()
            for pt in range(CP):
                for cc in range(DIM // 16):
                    vals = [buf_v[slot, pt * KP + j, pl.ds(cc * 16, 16)]
                            for j in range(KNN)]
                    vals.append(bias_v[pl.ds(cc * 16, 16)])
                    while len(vals) > 1:
                        nxt = [vals[i] + vals[i + 1]
                               for i in range(0, len(vals) - 1, 2)]
                        if len(vals) % 2:
                            nxt.append(vals[-1])
                        vals = nxt
                    ob_v[slot, pt, pl.ds(cc * 16, 16)] = vals[0]
            pltpu.async_copy(ob_v.at[slot], out_hbm.at[pl.ds(p0, CP)],
                             so[slot])

        for s in range(NB - 1):
            g_issue(s, s)

        def group(g, carry):
            for s in range(NB):
                c = NB * g + s

                @pl.when(c + NB - 1 < nchunks)
                def _():
                    g_issue((s + NB - 1) % NB, c + NB - 1)
                g_wait(s, c)
                process(s, c)
            return carry

        lax.fori_loop(0, ngroups, group, 0)
        for s in range(NB):
            c = nchunks - NB + s
            pltpu.make_async_copy(
                ob_v.at[s], out_hbm.at[pl.ds(pstart + c * CP, CP)],
                so[s]).wait()

    return body(z3, idsf, bias)


def kernel(x, x_v, W, b):
    nb, npts, dim = x.shape
    xvp = jnp.pad(x_v, ((0, 0), (0, 0), (0, 5)))
    xvt_t = jnp.swapaxes(xvp, 1, 2)
    ids = _topk(xvp, xvt_t)
    w2 = W.reshape(DIM, KNN, DIM).transpose(2, 1, 0).reshape(DIM, KNN * DIM)
    z = _zmat(x, w2)
    z3 = z.reshape(nb * npts * KNN, dim)
    idsf = ids.reshape(nb * npts * KP)
    out = _gather_sum(z3, idsf, b, nb * npts)
    return out.reshape(nb, npts, dim)
